# baseline (speedup 1.0000x reference)
import functools

import jax
import jax.numpy as jnp
from jax import lax
from jax.experimental import pallas as pl
from jax.experimental.pallas import tpu as pltpu
from jax.experimental.pallas import tpu_sc as plsc

VOCAB = 400000 + 1
DIM = 100
DIM_PAD = 104
BATCH = 4096 * 200

NUM_CORES = 2
NUM_SUBCORES = 16
NW = NUM_CORES * NUM_SUBCORES
B_PER_W = BATCH // NW

CHUNK = 128
N_CHUNKS = B_PER_W // CHUNK


@functools.cache
def _build_gather():
    mesh = plsc.VectorSubcoreMesh(
        core_axis_name="c", subcore_axis_name="s",
        num_cores=NUM_CORES, num_subcores=NUM_SUBCORES,
    )

    @functools.partial(
        pl.kernel,
        mesh=mesh,
        out_type=jax.ShapeDtypeStruct((BATCH, DIM_PAD), jnp.float32),
        scratch_types=[
            pltpu.VMEM((N_CHUNKS, CHUNK), jnp.int32),
            pltpu.VMEM((CHUNK, DIM_PAD), jnp.float32),
            pltpu.SemaphoreType.DMA,
        ],
        compiler_params=pltpu.CompilerParams(use_tc_tiling_on_sc=False),
    )
    def _gather_rows(idx_hbm, table_hbm, out_hbm, idx_v, rows_v, sem):
        wid = lax.axis_index("s") * NUM_CORES + lax.axis_index("c")
        base = wid * B_PER_W
        pltpu.sync_copy(idx_hbm.at[pl.ds(wid * N_CHUNKS, N_CHUNKS)], idx_v)

        def body(i, carry):
            pltpu.async_copy(table_hbm.at[idx_v.at[i]], rows_v, sem).wait()
            pltpu.sync_copy(rows_v, out_hbm.at[pl.ds(base + i * CHUNK, CHUNK)])
            return carry

        lax.fori_loop(0, N_CHUNKS, body, 0)

    return _gather_rows


def kernel(x, table):
    idx = x.reshape(BATCH // CHUNK, CHUNK).astype(jnp.int32)
    table_pad = jnp.pad(table, ((0, 0), (0, DIM_PAD - DIM)))
    out = _build_gather()(idx, table_pad)
    return out[:, :DIM].reshape(x.shape + (DIM,))


# speedup vs baseline: 1.1345x; 1.1345x over previous
import functools

import jax
import jax.numpy as jnp
from jax import lax
from jax.experimental import pallas as pl
from jax.experimental.pallas import tpu as pltpu
from jax.experimental.pallas import tpu_sc as plsc

VOCAB = 400000 + 1
DIM = 100
BATCH = 4096 * 200

NUM_CORES = 2
NUM_SUBCORES = 16
NW = NUM_CORES * NUM_SUBCORES

VBLOCKS = VOCAB // 128
K1_STEPS = -(-VBLOCKS // NW)

NBT = 4096 // 128
NST = 200 // 8


def _mesh():
    return plsc.VectorSubcoreMesh(
        core_axis_name="c", subcore_axis_name="s",
        num_cores=NUM_CORES, num_subcores=NUM_SUBCORES,
    )


@functools.cache
def _build_k1():
    @functools.partial(
        pl.kernel,
        mesh=_mesh(),
        out_type=jax.ShapeDtypeStruct((VOCAB + 7, 128), jnp.float32),
        scratch_types=[
            pltpu.VMEM((96, 128), jnp.float32),
            pltpu.VMEM((8, 128), jnp.float32),
            pltpu.VMEM((128, 128), jnp.float32),
        ],
        compiler_params=pltpu.CompilerParams(use_tc_tiling_on_sc=True, needs_layout_passes=False),
    )
    def k1(tT_hbm, tail8_hbm, tpad_hbm, tiles_v, tail_v, trows_v):
        wid = lax.axis_index("s") * NUM_CORES + lax.axis_index("c")
        viota = lax.iota(jnp.int32, 16)
        dvecs = [viota + 16 * m for m in range(6)]
        tail_rows = jnp.minimum(viota + 4, 7)

        def body(k, carry):
            c = wid + NW * k

            @pl.when(c < VBLOCKS)
            def _():
                for t in range(12):
                    pltpu.sync_copy(
                        tT_hbm.at[pl.ds(8 * t, 8), pl.ds(128 * c, 128)],
                        tiles_v.at[pl.ds(8 * t, 8)])
                pltpu.sync_copy(tail8_hbm.at[:, pl.ds(128 * c, 128)], tail_v)

                def trans(r, carry2):
                    rot = (r + viota) & 127
                    for m in range(6):
                        val = plsc.load_gather(tiles_v, [dvecs[m], rot])
                        plsc.store_scatter(trows_v, [rot, dvecs[m]], val)
                    val = plsc.load_gather(tail_v, [tail_rows, rot])
                    plsc.store_scatter(trows_v, [rot, viota + 96], val)
                    return carry2

                lax.fori_loop(0, 128, trans, 0)
                pltpu.sync_copy(trows_v, tpad_hbm.at[pl.ds(128 * c, 128)])

            return carry

        lax.fori_loop(0, K1_STEPS, body, 0)

        @pl.when(wid == NW - 1)
        def _():
            zero = jnp.zeros((16,), jnp.float32)

            def zrow(r, carry2):
                for j in range(8):
                    trows_v[r, pl.ds(16 * j, 16)] = zero
                return carry2

            lax.fori_loop(0, 8, zrow, 0)
            pltpu.sync_copy(trows_v.at[pl.ds(0, 8)],
                            tpad_hbm.at[pl.ds(VOCAB - 1, 8)])

    return k1


@functools.cache
def _build_k2():
    @functools.partial(
        pl.kernel,
        mesh=_mesh(),
        out_type=jax.ShapeDtypeStruct((BATCH, 128), jnp.float32),
        scratch_types=[
            pltpu.VMEM((8, 128), jnp.int32),
            pltpu.VMEM((8, 128), jnp.int32),
            pltpu.VMEM((2, 128, 128), jnp.float32),
            pltpu.SemaphoreType.DMA,
            pltpu.SemaphoreType.DMA,
        ],
        compiler_params=pltpu.CompilerParams(use_tc_tiling_on_sc=True, needs_layout_passes=False),
    )
    def k2(xT_hbm, tpad_hbm, out_hbm, xtile_v, xt_t, rows_q, gsem, wsem):
        wid = lax.axis_index("s") * NUM_CORES + lax.axis_index("c")
        bt = wid
        viota = lax.iota(jnp.int32, 16)
        src_row = viota % 8
        src_col_base = viota // 8

        def body(st, carry):
            pltpu.sync_copy(
                xT_hbm.at[pl.ds(8 * st, 8), pl.ds(128 * bt, 128)], xtile_v)

            def tloop(k, carry2):
                for j in range(8):
                    col = src_col_base + (16 * k + 2 * j)
                    val = plsc.load_gather(xtile_v, [src_row, col])
                    xt_t[k, pl.ds(16 * j, 16)] = val
                return carry2

            lax.fori_loop(0, 8, tloop, 0)

            def gloop(k, carry2):
                buf = k % 2
                pltpu.async_copy(
                    tpad_hbm.at[xt_t.at[k]], rows_q.at[buf], gsem).wait()
                for b2 in range(16):
                    row0 = (128 * bt + 16 * k + b2) * 200 + 8 * st
                    pltpu.async_copy(
                        rows_q.at[buf, pl.ds(8 * b2, 8)],
                        out_hbm.at[pl.ds(row0, 8)], wsem)
                for b2 in range(16):
                    pltpu.make_async_copy(
                        rows_q.at[buf, pl.ds(0, 8)],
                        out_hbm.at[pl.ds(0, 8)], wsem).wait()
                return carry2

            lax.fori_loop(0, 8, gloop, 0)
            return carry

        lax.fori_loop(0, NST, body, 0)

    return k2


def kernel(x, table):
    tT = table.T
    tail8 = table[:, 92:100].T
    xT = x.T.astype(jnp.int32)
    tpad = _build_k1()(tT, tail8)
    outp = _build_k2()(xT, tpad)
    out = outp[:, :DIM]
    return out.reshape(4096, 200, DIM)


# speedup vs baseline: 1.9418x; 1.7116x over previous
import functools

import jax
import jax.numpy as jnp
from jax import lax
from jax.experimental import pallas as pl
from jax.experimental.pallas import tpu as pltpu
from jax.experimental.pallas import tpu_sc as plsc

VOCAB = 400000 + 1
DIM = 100
BATCH = 4096 * 200

NUM_CORES = 2
NUM_SUBCORES = 16
NW = NUM_CORES * NUM_SUBCORES

VBLOCKS = VOCAB // 128
K1_STEPS = -(-VBLOCKS // NW)

NBT = 4096 // 128
NST = 200 // 8


def _mesh():
    return plsc.VectorSubcoreMesh(
        core_axis_name="c", subcore_axis_name="s",
        num_cores=NUM_CORES, num_subcores=NUM_SUBCORES,
    )


@functools.cache
def _build_k1():
    @functools.partial(
        pl.kernel,
        mesh=_mesh(),
        out_type=jax.ShapeDtypeStruct((VOCAB + 7, 128), jnp.float32),
        scratch_types=[
            pltpu.VMEM((2, 96, 128), jnp.float32),
            pltpu.VMEM((2, 8, 128), jnp.float32),
            pltpu.VMEM((2, 128, 128), jnp.float32),
            pltpu.SemaphoreType.DMA,
            pltpu.SemaphoreType.DMA,
        ],
        compiler_params=pltpu.CompilerParams(use_tc_tiling_on_sc=True, needs_layout_passes=False),
    )
    def k1(tT_hbm, tail8_hbm, tpad_hbm, tiles_v, tail_v, trows_v, rsem, wsem):
        wid = lax.axis_index("s") * NUM_CORES + lax.axis_index("c")
        viota = lax.iota(jnp.int32, 16)
        dvecs = [viota + 16 * m for m in range(6)]
        tail_rows = jnp.minimum(viota + 4, 7)

        def fire_reads(c, buf):
            for t in range(12):
                pltpu.async_copy(
                    tT_hbm.at[pl.ds(8 * t, 8), pl.ds(128 * c, 128)],
                    tiles_v.at[buf, pl.ds(8 * t, 8)], rsem)
            pltpu.async_copy(tail8_hbm.at[:, pl.ds(128 * c, 128)],
                             tail_v.at[buf], rsem)

        def drain_reads(buf):
            for t in range(12):
                pltpu.make_async_copy(
                    tT_hbm.at[pl.ds(0, 8), pl.ds(0, 128)],
                    tiles_v.at[buf, pl.ds(0, 8)], rsem).wait()
            pltpu.make_async_copy(tail8_hbm.at[:, pl.ds(0, 128)],
                                  tail_v.at[buf], rsem).wait()

        fire_reads(wid, 0)

        def body(k, carry):
            c = wid + NW * k
            cn = c + NW
            buf = k % 2

            @pl.when(cn < VBLOCKS)
            def _():
                fire_reads(cn, 1 - buf)

            @pl.when(c < VBLOCKS)
            def _():
                drain_reads(buf)

                def trans(r, carry2):
                    rot = (r + viota) & 127
                    for m in range(6):
                        val = plsc.load_gather(tiles_v.at[buf], [dvecs[m], rot])
                        plsc.store_scatter(trows_v.at[buf], [rot, dvecs[m]], val)
                    val = plsc.load_gather(tail_v.at[buf], [tail_rows, rot])
                    plsc.store_scatter(trows_v.at[buf], [rot, viota + 96], val)
                    return carry2

                @pl.when(k >= 2)
                def _():
                    pltpu.make_async_copy(
                        tpad_hbm.at[pl.ds(0, 128)],
                        trows_v.at[buf], wsem).wait()

                lax.fori_loop(0, 128, trans, 0)
                pltpu.async_copy(trows_v.at[buf],
                                 tpad_hbm.at[pl.ds(128 * c, 128)], wsem)

            return carry

        lax.fori_loop(0, K1_STEPS, body, 0)

        for _ in range(2):
            pltpu.make_async_copy(
                tpad_hbm.at[pl.ds(0, 128)],
                trows_v.at[0], wsem).wait()

        @pl.when(wid == NW - 1)
        def _():
            zero = jnp.zeros((16,), jnp.float32)

            def zrow(r, carry2):
                for j in range(8):
                    trows_v[0, r, pl.ds(16 * j, 16)] = zero
                return carry2

            lax.fori_loop(0, 8, zrow, 0)
            pltpu.sync_copy(trows_v.at[0, pl.ds(0, 8)],
                            tpad_hbm.at[pl.ds(VOCAB - 1, 8)])

    return k1


@functools.cache
def _build_k2():
    @functools.partial(
        pl.kernel,
        mesh=_mesh(),
        out_type=jax.ShapeDtypeStruct((BATCH, 128), jnp.float32),
        scratch_types=[
            pltpu.VMEM((8, 128), jnp.int32),
            pltpu.VMEM((8, 128), jnp.int32),
            pltpu.VMEM((2, 128, 128), jnp.float32),
            pltpu.SemaphoreType.DMA,
            pltpu.SemaphoreType.DMA,
        ],
        compiler_params=pltpu.CompilerParams(use_tc_tiling_on_sc=True, needs_layout_passes=False),
    )
    def k2(xT_hbm, tpad_hbm, out_hbm, xtile_v, xt_t, rows_q, gsem, wsem):
        wid = lax.axis_index("s") * NUM_CORES + lax.axis_index("c")
        bt = wid
        viota = lax.iota(jnp.int32, 16)
        src_row = viota % 8
        src_col_base = viota // 8

        def body(st, carry):
            pltpu.sync_copy(
                xT_hbm.at[pl.ds(8 * st, 8), pl.ds(128 * bt, 128)], xtile_v)

            def tloop(k, carry2):
                for j in range(8):
                    col = src_col_base + (16 * k + 2 * j)
                    val = plsc.load_gather(xtile_v, [src_row, col])
                    xt_t[k, pl.ds(16 * j, 16)] = val
                return carry2

            lax.fori_loop(0, 8, tloop, 0)

            def gloop(k, carry2):
                buf = k % 2
                pltpu.async_copy(
                    tpad_hbm.at[xt_t.at[k]], rows_q.at[buf], gsem).wait()
                for b2 in range(16):
                    row0 = (128 * bt + 16 * k + b2) * 200 + 8 * st
                    pltpu.async_copy(
                        rows_q.at[buf, pl.ds(8 * b2, 8)],
                        out_hbm.at[pl.ds(row0, 8)], wsem)
                for b2 in range(16):
                    pltpu.make_async_copy(
                        rows_q.at[buf, pl.ds(0, 8)],
                        out_hbm.at[pl.ds(0, 8)], wsem).wait()
                return carry2

            lax.fori_loop(0, 8, gloop, 0)
            return carry

        lax.fori_loop(0, NST, body, 0)

    return k2


def kernel(x, table):
    tT = table.T
    tail8 = table[:, 92:100].T
    xT = x.T.astype(jnp.int32)
    tpad = _build_k1()(tT, tail8)
    outp = _build_k2()(xT, tpad)
    out = outp[:, :DIM]
    return out.reshape(4096, 200, DIM)


# speedup vs baseline: 2.0354x; 1.0482x over previous
import functools

import jax
import jax.numpy as jnp
from jax import lax
from jax.experimental import pallas as pl
from jax.experimental.pallas import tpu as pltpu
from jax.experimental.pallas import tpu_sc as plsc

VOCAB = 400000 + 1
DIM = 100
BATCH = 4096 * 200

NUM_CORES = 2
NUM_SUBCORES = 16
NW = NUM_CORES * NUM_SUBCORES

VBLOCKS = VOCAB // 128
K1_STEPS = -(-VBLOCKS // NW)

NBT = 4096 // 128
NST = 200 // 8


def _mesh():
    return plsc.VectorSubcoreMesh(
        core_axis_name="c", subcore_axis_name="s",
        num_cores=NUM_CORES, num_subcores=NUM_SUBCORES,
    )


@functools.cache
def _build_k1():
    @functools.partial(
        pl.kernel,
        mesh=_mesh(),
        out_type=jax.ShapeDtypeStruct((VOCAB + 7, 128), jnp.float32),
        scratch_types=[
            pltpu.VMEM((2, 96, 128), jnp.float32),
            pltpu.VMEM((2, 8, 128), jnp.float32),
            pltpu.VMEM((2, 128, 128), jnp.float32),
            pltpu.SemaphoreType.DMA,
            pltpu.SemaphoreType.DMA,
        ],
        compiler_params=pltpu.CompilerParams(use_tc_tiling_on_sc=True, needs_layout_passes=False),
    )
    def k1(tT_hbm, tail8_hbm, tpad_hbm, tiles_v, tail_v, trows_v, rsem, wsem):
        wid = lax.axis_index("s") * NUM_CORES + lax.axis_index("c")
        viota = lax.iota(jnp.int32, 16)
        dvecs = [viota + 16 * m for m in range(6)]
        tail_rows = jnp.minimum(viota + 4, 7)

        def fire_reads(c, buf):
            for t in range(12):
                pltpu.async_copy(
                    tT_hbm.at[pl.ds(8 * t, 8), pl.ds(128 * c, 128)],
                    tiles_v.at[buf, pl.ds(8 * t, 8)], rsem)
            pltpu.async_copy(tail8_hbm.at[:, pl.ds(128 * c, 128)],
                             tail_v.at[buf], rsem)

        def drain_reads(buf):
            for t in range(12):
                pltpu.make_async_copy(
                    tT_hbm.at[pl.ds(0, 8), pl.ds(0, 128)],
                    tiles_v.at[buf, pl.ds(0, 8)], rsem).wait()
            pltpu.make_async_copy(tail8_hbm.at[:, pl.ds(0, 128)],
                                  tail_v.at[buf], rsem).wait()

        fire_reads(wid, 0)

        def body(k, carry):
            c = wid + NW * k
            cn = c + NW
            buf = k % 2

            @pl.when(cn < VBLOCKS)
            def _():
                fire_reads(cn, 1 - buf)

            @pl.when(c < VBLOCKS)
            def _():
                drain_reads(buf)

                def trans(r, carry2):
                    rot = (r + viota) & 127
                    for m in range(6):
                        val = plsc.load_gather(tiles_v.at[buf], [dvecs[m], rot])
                        plsc.store_scatter(trows_v.at[buf], [rot, dvecs[m]], val)
                    val = plsc.load_gather(tail_v.at[buf], [tail_rows, rot])
                    plsc.store_scatter(trows_v.at[buf], [rot, viota + 96], val)
                    return carry2

                @pl.when(k >= 2)
                def _():
                    pltpu.make_async_copy(
                        tpad_hbm.at[pl.ds(0, 128)],
                        trows_v.at[buf], wsem).wait()

                lax.fori_loop(0, 128, trans, 0)
                pltpu.async_copy(trows_v.at[buf],
                                 tpad_hbm.at[pl.ds(128 * c, 128)], wsem)

            return carry

        lax.fori_loop(0, K1_STEPS, body, 0)

        for _ in range(2):
            pltpu.make_async_copy(
                tpad_hbm.at[pl.ds(0, 128)],
                trows_v.at[0], wsem).wait()

        @pl.when(wid == NW - 1)
        def _():
            zero = jnp.zeros((16,), jnp.float32)

            def zrow(r, carry2):
                for j in range(8):
                    trows_v[0, r, pl.ds(16 * j, 16)] = zero
                return carry2

            lax.fori_loop(0, 8, zrow, 0)
            pltpu.sync_copy(trows_v.at[0, pl.ds(0, 8)],
                            tpad_hbm.at[pl.ds(VOCAB - 1, 8)])

    return k1


@functools.cache
def _build_k2():
    @functools.partial(
        pl.kernel,
        mesh=_mesh(),
        out_type=jax.ShapeDtypeStruct((BATCH, 128), jnp.float32),
        scratch_types=[
            pltpu.VMEM((8, 128), jnp.int32),
            pltpu.VMEM((8, 128), jnp.int32),
            pltpu.VMEM((2, 128, 128), jnp.float32),
            pltpu.SemaphoreType.DMA,
            pltpu.SemaphoreType.DMA,
        ],
        compiler_params=pltpu.CompilerParams(use_tc_tiling_on_sc=True, needs_layout_passes=False),
    )
    def k2(xT_hbm, tpad_hbm, out_hbm, xtile_v, xt_t, rows_q, gsem, wsem):
        wid = lax.axis_index("s") * NUM_CORES + lax.axis_index("c")
        bt = wid
        viota = lax.iota(jnp.int32, 16)
        src_row = viota % 8
        src_col_base = viota // 8

        def body(st, carry):
            pltpu.sync_copy(
                xT_hbm.at[pl.ds(8 * st, 8), pl.ds(128 * bt, 128)], xtile_v)

            def tloop(k, carry2):
                for j in range(8):
                    col = src_col_base + (16 * k + 2 * j)
                    val = plsc.load_gather(xtile_v, [src_row, col])
                    xt_t[k, pl.ds(16 * j, 16)] = val
                return carry2

            lax.fori_loop(0, 8, tloop, 0)

            def drain_writes():
                for _ in range(16):
                    pltpu.make_async_copy(
                        rows_q.at[0, pl.ds(0, 8)],
                        out_hbm.at[pl.ds(0, 8)], wsem).wait()

            pltpu.async_copy(tpad_hbm.at[xt_t.at[0]], rows_q.at[0], gsem)
            for k in range(8):
                buf = k % 2
                pltpu.make_async_copy(
                    tpad_hbm.at[xt_t.at[0]], rows_q.at[buf], gsem).wait()
                if k + 1 < 8:
                    if k >= 1:
                        drain_writes()
                    pltpu.async_copy(
                        tpad_hbm.at[xt_t.at[k + 1]],
                        rows_q.at[1 - buf], gsem)
                for b2 in range(16):
                    row0 = (128 * bt + 16 * k + b2) * 200 + 8 * st
                    pltpu.async_copy(
                        rows_q.at[buf, pl.ds(8 * b2, 8)],
                        out_hbm.at[pl.ds(row0, 8)], wsem)
            drain_writes()
            drain_writes()
            return carry

        lax.fori_loop(0, NST, body, 0)

    return k2


def kernel(x, table):
    tT = table.T
    tail8 = table[:, 92:100].T
    xT = x.T.astype(jnp.int32)
    tpad = _build_k1()(tT, tail8)
    outp = _build_k2()(xT, tpad)
    out = outp[:, :DIM]
    return out.reshape(4096, 200, DIM)


# speedup vs baseline: 2.0702x; 1.0171x over previous
import functools

import jax
import jax.numpy as jnp
from jax import lax
from jax.experimental import pallas as pl
from jax.experimental.pallas import tpu as pltpu
from jax.experimental.pallas import tpu_sc as plsc

VOCAB = 400000 + 1
DIM = 100
BATCH = 4096 * 200

NUM_CORES = 2
NUM_SUBCORES = 16
NW = NUM_CORES * NUM_SUBCORES

VBLOCKS = VOCAB // 128
K1_STEPS = -(-VBLOCKS // NW)

NBT = 4096 // 128
NST = 200 // 8


def _mesh():
    return plsc.VectorSubcoreMesh(
        core_axis_name="c", subcore_axis_name="s",
        num_cores=NUM_CORES, num_subcores=NUM_SUBCORES,
    )


@functools.cache
def _build_k1():
    @functools.partial(
        pl.kernel,
        mesh=_mesh(),
        out_type=jax.ShapeDtypeStruct((VOCAB + 7, 128), jnp.float32),
        scratch_types=[
            pltpu.VMEM((2, 96, 128), jnp.float32),
            pltpu.VMEM((2, 8, 128), jnp.float32),
            pltpu.VMEM((2, 128, 128), jnp.float32),
            pltpu.SemaphoreType.DMA,
            pltpu.SemaphoreType.DMA,
        ],
        compiler_params=pltpu.CompilerParams(use_tc_tiling_on_sc=True, needs_layout_passes=False),
    )
    def k1(tT_hbm, tail8_hbm, tpad_hbm, tiles_v, tail_v, trows_v, rsem, wsem):
        wid = lax.axis_index("s") * NUM_CORES + lax.axis_index("c")
        viota = lax.iota(jnp.int32, 16)
        dvecs = [viota + 16 * m for m in range(6)]
        tail_rows = jnp.minimum(viota + 4, 7)

        def fire_reads(c, buf):
            for t in range(12):
                pltpu.async_copy(
                    tT_hbm.at[pl.ds(8 * t, 8), pl.ds(128 * c, 128)],
                    tiles_v.at[buf, pl.ds(8 * t, 8)], rsem)
            pltpu.async_copy(tail8_hbm.at[:, pl.ds(128 * c, 128)],
                             tail_v.at[buf], rsem)

        def drain_reads(buf):
            pltpu.make_async_copy(
                tT_hbm.at[pl.ds(0, 96), pl.ds(0, 128)],
                tiles_v.at[buf], rsem).wait()
            pltpu.make_async_copy(tail8_hbm.at[:, pl.ds(0, 128)],
                                  tail_v.at[buf], rsem).wait()

        fire_reads(wid, 0)

        def body(k, carry):
            c = wid + NW * k
            cn = c + NW
            buf = k % 2

            @pl.when(cn < VBLOCKS)
            def _():
                fire_reads(cn, 1 - buf)

            @pl.when(c < VBLOCKS)
            def _():
                drain_reads(buf)

                def trans(r, carry2):
                    rot = (r + viota) & 127
                    for m in range(6):
                        val = plsc.load_gather(tiles_v.at[buf], [dvecs[m], rot])
                        plsc.store_scatter(trows_v.at[buf], [rot, dvecs[m]], val)
                    val = plsc.load_gather(tail_v.at[buf], [tail_rows, rot])
                    plsc.store_scatter(trows_v.at[buf], [rot, viota + 96], val)
                    return carry2

                @pl.when(k >= 2)
                def _():
                    pltpu.make_async_copy(
                        tpad_hbm.at[pl.ds(0, 128)],
                        trows_v.at[buf], wsem).wait()

                lax.fori_loop(0, 128, trans, 0)
                pltpu.async_copy(trows_v.at[buf],
                                 tpad_hbm.at[pl.ds(128 * c, 128)], wsem)

            return carry

        lax.fori_loop(0, K1_STEPS, body, 0)

        for _ in range(2):
            pltpu.make_async_copy(
                tpad_hbm.at[pl.ds(0, 128)],
                trows_v.at[0], wsem).wait()

        @pl.when(wid == NW - 1)
        def _():
            zero = jnp.zeros((16,), jnp.float32)

            def zrow(r, carry2):
                for j in range(8):
                    trows_v[0, r, pl.ds(16 * j, 16)] = zero
                return carry2

            lax.fori_loop(0, 8, zrow, 0)
            pltpu.sync_copy(trows_v.at[0, pl.ds(0, 8)],
                            tpad_hbm.at[pl.ds(VOCAB - 1, 8)])

    return k1


@functools.cache
def _build_k2():
    @functools.partial(
        pl.kernel,
        mesh=_mesh(),
        out_type=jax.ShapeDtypeStruct((BATCH, 128), jnp.float32),
        scratch_types=[
            pltpu.VMEM((8, 128), jnp.int32),
            pltpu.VMEM((8, 128), jnp.int32),
            pltpu.VMEM((2, 128, 128), jnp.float32),
            pltpu.SemaphoreType.DMA,
            pltpu.SemaphoreType.DMA,
        ],
        compiler_params=pltpu.CompilerParams(use_tc_tiling_on_sc=True, needs_layout_passes=False),
    )
    def k2(xT_hbm, tpad_hbm, out_hbm, xtile_v, xt_t, rows_q, gsem, wsem):
        wid = lax.axis_index("s") * NUM_CORES + lax.axis_index("c")
        bt = wid
        viota = lax.iota(jnp.int32, 16)
        src_row = viota % 8
        src_col_base = viota // 8

        def body(st, carry):
            pltpu.sync_copy(
                xT_hbm.at[pl.ds(8 * st, 8), pl.ds(128 * bt, 128)], xtile_v)

            def tloop(k, carry2):
                for j in range(8):
                    col = src_col_base + (16 * k + 2 * j)
                    val = plsc.load_gather(xtile_v, [src_row, col])
                    xt_t[k, pl.ds(16 * j, 16)] = val
                return carry2

            lax.fori_loop(0, 8, tloop, 0)

            def drain_writes():
                pltpu.make_async_copy(
                    rows_q.at[0], out_hbm.at[pl.ds(0, 128)], wsem).wait()

            pltpu.async_copy(tpad_hbm.at[xt_t.at[0]], rows_q.at[0], gsem)
            for k in range(8):
                buf = k % 2
                pltpu.make_async_copy(
                    tpad_hbm.at[xt_t.at[0]], rows_q.at[buf], gsem).wait()
                if k + 1 < 8:
                    if k >= 1:
                        drain_writes()
                    pltpu.async_copy(
                        tpad_hbm.at[xt_t.at[k + 1]],
                        rows_q.at[1 - buf], gsem)
                for b2 in range(16):
                    row0 = (128 * bt + 16 * k + b2) * 200 + 8 * st
                    pltpu.async_copy(
                        rows_q.at[buf, pl.ds(8 * b2, 8)],
                        out_hbm.at[pl.ds(row0, 8)], wsem)
            drain_writes()
            drain_writes()
            return carry

        lax.fori_loop(0, NST, body, 0)

    return k2


def kernel(x, table):
    tT = table.T
    tail8 = table[:, 92:100].T
    xT = x.T.astype(jnp.int32)
    tpad = _build_k1()(tT, tail8)
    outp = _build_k2()(xT, tpad)
    out = outp[:, :DIM]
    return out.reshape(4096, 200, DIM)


# speedup vs baseline: 2.2286x; 1.0765x over previous
import functools

import jax
import jax.numpy as jnp
from jax import lax
from jax.experimental import pallas as pl
from jax.experimental.pallas import tpu as pltpu
from jax.experimental.pallas import tpu_sc as plsc

VOCAB = 400000 + 1
DIM = 100
BATCH = 4096 * 200

NUM_CORES = 2
NUM_SUBCORES = 16
NW = NUM_CORES * NUM_SUBCORES

VBLOCKS = VOCAB // 128
K1_STEPS = -(-VBLOCKS // NW)

NBT = 4096 // 128
NST = 200 // 8


def _mesh():
    return plsc.VectorSubcoreMesh(
        core_axis_name="c", subcore_axis_name="s",
        num_cores=NUM_CORES, num_subcores=NUM_SUBCORES,
    )


@functools.cache
def _build_k1():
    @functools.partial(
        pl.kernel,
        mesh=_mesh(),
        out_type=jax.ShapeDtypeStruct((VOCAB + 7, 128), jnp.float32),
        scratch_types=[
            pltpu.VMEM((2, 96, 128), jnp.float32),
            pltpu.VMEM((2, 8, 128), jnp.float32),
            pltpu.VMEM((2, 128, 128), jnp.float32),
            pltpu.SemaphoreType.DMA((2,)),
            pltpu.SemaphoreType.DMA,
        ],
        compiler_params=pltpu.CompilerParams(use_tc_tiling_on_sc=True, needs_layout_passes=False),
    )
    def k1(tT_hbm, tail8_hbm, tpad_hbm, tiles_v, tail_v, trows_v, rsem, wsem):
        wid = lax.axis_index("s") * NUM_CORES + lax.axis_index("c")
        viota = lax.iota(jnp.int32, 16)
        dvecs = [viota + 16 * m for m in range(6)]
        tail_rows = jnp.minimum(viota + 4, 7)

        def fire_reads(c, buf):
            for t in range(12):
                pltpu.async_copy(
                    tT_hbm.at[pl.ds(8 * t, 8), pl.ds(128 * c, 128)],
                    tiles_v.at[buf, pl.ds(8 * t, 8)], rsem.at[buf])
            pltpu.async_copy(tail8_hbm.at[:, pl.ds(128 * c, 128)],
                             tail_v.at[buf], rsem.at[buf])

        def drain_reads(buf):
            pltpu.make_async_copy(
                tT_hbm.at[pl.ds(0, 96), pl.ds(0, 128)],
                tiles_v.at[buf], rsem.at[buf]).wait()
            pltpu.make_async_copy(tail8_hbm.at[:, pl.ds(0, 128)],
                                  tail_v.at[buf], rsem.at[buf]).wait()

        fire_reads(wid, 0)

        def body(k, carry):
            c = wid + NW * k
            cn = c + NW
            buf = k % 2

            @pl.when(cn < VBLOCKS)
            def _():
                fire_reads(cn, 1 - buf)

            @pl.when(c < VBLOCKS)
            def _():
                drain_reads(buf)

                def trans(r, carry2):
                    rot = (r + viota) & 127
                    for m in range(6):
                        val = plsc.load_gather(tiles_v.at[buf], [dvecs[m], rot])
                        plsc.store_scatter(trows_v.at[buf], [rot, dvecs[m]], val)
                    val = plsc.load_gather(tail_v.at[buf], [tail_rows, rot])
                    plsc.store_scatter(trows_v.at[buf], [rot, viota + 96], val)
                    return carry2

                @pl.when(k >= 2)
                def _():
                    pltpu.make_async_copy(
                        tpad_hbm.at[pl.ds(0, 128)],
                        trows_v.at[buf], wsem).wait()

                lax.fori_loop(0, 128, trans, 0)
                pltpu.async_copy(trows_v.at[buf],
                                 tpad_hbm.at[pl.ds(128 * c, 128)], wsem)

            return carry

        lax.fori_loop(0, K1_STEPS, body, 0)

        for _ in range(2):
            pltpu.make_async_copy(
                tpad_hbm.at[pl.ds(0, 128)],
                trows_v.at[0], wsem).wait()

        @pl.when(wid == NW - 1)
        def _():
            zero = jnp.zeros((16,), jnp.float32)

            def zrow(r, carry2):
                for j in range(8):
                    trows_v[0, r, pl.ds(16 * j, 16)] = zero
                return carry2

            lax.fori_loop(0, 8, zrow, 0)
            pltpu.sync_copy(trows_v.at[0, pl.ds(0, 8)],
                            tpad_hbm.at[pl.ds(VOCAB - 1, 8)])

    return k1


@functools.cache
def _build_k2():
    @functools.partial(
        pl.kernel,
        mesh=_mesh(),
        out_type=jax.ShapeDtypeStruct((BATCH, 128), jnp.float32),
        scratch_types=[
            pltpu.VMEM((8, 128), jnp.int32),
            pltpu.VMEM((8, 128), jnp.int32),
            pltpu.VMEM((2, 128, 128), jnp.float32),
            pltpu.SemaphoreType.DMA((2,)),
            pltpu.SemaphoreType.DMA,
        ],
        compiler_params=pltpu.CompilerParams(use_tc_tiling_on_sc=True, needs_layout_passes=False),
    )
    def k2(xT_hbm, tpad_hbm, out_hbm, xtile_v, xt_t, rows_q, gsem, wsem):
        wid = lax.axis_index("s") * NUM_CORES + lax.axis_index("c")
        bt = wid
        viota = lax.iota(jnp.int32, 16)
        src_row = viota % 8
        src_col_base = viota // 8

        def body(st, carry):
            pltpu.sync_copy(
                xT_hbm.at[pl.ds(8 * st, 8), pl.ds(128 * bt, 128)], xtile_v)

            def tloop(k, carry2):
                for j in range(8):
                    col = src_col_base + (16 * k + 2 * j)
                    val = plsc.load_gather(xtile_v, [src_row, col])
                    xt_t[k, pl.ds(16 * j, 16)] = val
                return carry2

            lax.fori_loop(0, 8, tloop, 0)

            def drain_writes():
                pltpu.make_async_copy(
                    rows_q.at[0], out_hbm.at[pl.ds(0, 128)], wsem).wait()

            pltpu.async_copy(tpad_hbm.at[xt_t.at[0]], rows_q.at[0],
                             gsem.at[0])
            for k in range(8):
                buf = k % 2
                if k + 1 < 8:
                    if k >= 1:
                        drain_writes()
                    pltpu.async_copy(
                        tpad_hbm.at[xt_t.at[k + 1]],
                        rows_q.at[1 - buf], gsem.at[1 - buf])
                pltpu.make_async_copy(
                    tpad_hbm.at[xt_t.at[0]], rows_q.at[buf],
                    gsem.at[buf]).wait()
                for b2 in range(16):
                    row0 = (128 * bt + 16 * k + b2) * 200 + 8 * st
                    pltpu.async_copy(
                        rows_q.at[buf, pl.ds(8 * b2, 8)],
                        out_hbm.at[pl.ds(row0, 8)], wsem)
            drain_writes()
            drain_writes()
            return carry

        lax.fori_loop(0, NST, body, 0)

    return k2


def kernel(x, table):
    tT = table.T
    tail8 = table[:, 92:100].T
    xT = x.T.astype(jnp.int32)
    tpad = _build_k1()(tT, tail8)
    outp = _build_k2()(xT, tpad)
    out = outp[:, :DIM]
    return out.reshape(4096, 200, DIM)


# speedup vs baseline: 2.2391x; 1.0047x over previous
import functools

import jax
import jax.numpy as jnp
from jax import lax
from jax.experimental import pallas as pl
from jax.experimental.pallas import tpu as pltpu
from jax.experimental.pallas import tpu_sc as plsc

VOCAB = 400000 + 1
DIM = 100
BATCH = 4096 * 200

NUM_CORES = 2
NUM_SUBCORES = 16
NW = NUM_CORES * NUM_SUBCORES

VBLOCKS = VOCAB // 128
K1_STEPS = -(-VBLOCKS // NW)

NBT = 4096 // 128
NST = 200 // 8


def _mesh():
    return plsc.VectorSubcoreMesh(
        core_axis_name="c", subcore_axis_name="s",
        num_cores=NUM_CORES, num_subcores=NUM_SUBCORES,
    )


@functools.cache
def _build_k1():
    @functools.partial(
        pl.kernel,
        mesh=_mesh(),
        out_type=jax.ShapeDtypeStruct((VOCAB + 7, 128), jnp.float32),
        scratch_types=[
            pltpu.VMEM((2, 96, 128), jnp.float32),
            pltpu.VMEM((2, 8, 128), jnp.float32),
            pltpu.VMEM((2, 128, 128), jnp.float32),
            pltpu.SemaphoreType.DMA((2,)),
            pltpu.SemaphoreType.DMA,
        ],
        compiler_params=pltpu.CompilerParams(use_tc_tiling_on_sc=True, needs_layout_passes=False),
    )
    def k1(tT_hbm, tail8_hbm, tpad_hbm, tiles_v, tail_v, trows_v, rsem, wsem):
        wid = lax.axis_index("s") * NUM_CORES + lax.axis_index("c")
        viota = lax.iota(jnp.int32, 16)
        dvecs = [viota + 16 * m for m in range(6)]
        tail_rows = jnp.minimum(viota + 4, 7)

        def fire_reads(c, buf):
            for t in range(12):
                pltpu.async_copy(
                    tT_hbm.at[pl.ds(8 * t, 8), pl.ds(128 * c, 128)],
                    tiles_v.at[buf, pl.ds(8 * t, 8)], rsem.at[buf])
            pltpu.async_copy(tail8_hbm.at[:, pl.ds(128 * c, 128)],
                             tail_v.at[buf], rsem.at[buf])

        def drain_reads(buf):
            pltpu.make_async_copy(
                tT_hbm.at[pl.ds(0, 96), pl.ds(0, 128)],
                tiles_v.at[buf], rsem.at[buf]).wait()
            pltpu.make_async_copy(tail8_hbm.at[:, pl.ds(0, 128)],
                                  tail_v.at[buf], rsem.at[buf]).wait()

        fire_reads(wid, 0)

        def body(k, carry):
            c = wid + NW * k
            cn = c + NW
            buf = k % 2

            @pl.when(cn < VBLOCKS)
            def _():
                fire_reads(cn, 1 - buf)

            @pl.when(c < VBLOCKS)
            def _():
                drain_reads(buf)

                def trans(r2, carry2):
                    for h in range(2):
                        rot = (2 * r2 + h + viota) & 127
                        for m in range(6):
                            val = plsc.load_gather(
                                tiles_v.at[buf], [dvecs[m], rot])
                            plsc.store_scatter(
                                trows_v.at[buf], [rot, dvecs[m]], val)
                        val = plsc.load_gather(
                            tail_v.at[buf], [tail_rows, rot])
                        plsc.store_scatter(
                            trows_v.at[buf], [rot, viota + 96], val)
                    return carry2

                @pl.when(k >= 2)
                def _():
                    pltpu.make_async_copy(
                        tpad_hbm.at[pl.ds(0, 128)],
                        trows_v.at[buf], wsem).wait()

                lax.fori_loop(0, 64, trans, 0)
                pltpu.async_copy(trows_v.at[buf],
                                 tpad_hbm.at[pl.ds(128 * c, 128)], wsem)

            return carry

        lax.fori_loop(0, K1_STEPS, body, 0)

        for _ in range(2):
            pltpu.make_async_copy(
                tpad_hbm.at[pl.ds(0, 128)],
                trows_v.at[0], wsem).wait()

        @pl.when(wid == NW - 1)
        def _():
            zero = jnp.zeros((16,), jnp.float32)

            def zrow(r, carry2):
                for j in range(8):
                    trows_v[0, r, pl.ds(16 * j, 16)] = zero
                return carry2

            lax.fori_loop(0, 8, zrow, 0)
            pltpu.sync_copy(trows_v.at[0, pl.ds(0, 8)],
                            tpad_hbm.at[pl.ds(VOCAB - 1, 8)])

    return k1


@functools.cache
def _build_k2():
    @functools.partial(
        pl.kernel,
        mesh=_mesh(),
        out_type=jax.ShapeDtypeStruct((BATCH, 128), jnp.float32),
        scratch_types=[
            pltpu.VMEM((8, 128), jnp.int32),
            pltpu.VMEM((8, 128), jnp.int32),
            pltpu.VMEM((2, 128, 128), jnp.float32),
            pltpu.SemaphoreType.DMA((2,)),
            pltpu.SemaphoreType.DMA,
        ],
        compiler_params=pltpu.CompilerParams(use_tc_tiling_on_sc=True, needs_layout_passes=False),
    )
    def k2(xT_hbm, tpad_hbm, out_hbm, xtile_v, xt_t, rows_q, gsem, wsem):
        wid = lax.axis_index("s") * NUM_CORES + lax.axis_index("c")
        bt = wid
        viota = lax.iota(jnp.int32, 16)
        src_row = viota % 8
        src_col_base = viota // 8

        def body(st, carry):
            pltpu.sync_copy(
                xT_hbm.at[pl.ds(8 * st, 8), pl.ds(128 * bt, 128)], xtile_v)

            def tloop(k, carry2):
                for j in range(8):
                    col = src_col_base + (16 * k + 2 * j)
                    val = plsc.load_gather(xtile_v, [src_row, col])
                    xt_t[k, pl.ds(16 * j, 16)] = val
                return carry2

            lax.fori_loop(0, 8, tloop, 0)

            def drain_writes():
                pltpu.make_async_copy(
                    rows_q.at[0], out_hbm.at[pl.ds(0, 128)], wsem).wait()

            pltpu.async_copy(tpad_hbm.at[xt_t.at[0]], rows_q.at[0],
                             gsem.at[0])
            for k in range(8):
                buf = k % 2
                if k + 1 < 8:
                    if k >= 1:
                        drain_writes()
                    pltpu.async_copy(
                        tpad_hbm.at[xt_t.at[k + 1]],
                        rows_q.at[1 - buf], gsem.at[1 - buf])
                pltpu.make_async_copy(
                    tpad_hbm.at[xt_t.at[0]], rows_q.at[buf],
                    gsem.at[buf]).wait()
                for b2 in range(16):
                    row0 = (128 * bt + 16 * k + b2) * 200 + 8 * st
                    pltpu.async_copy(
                        rows_q.at[buf, pl.ds(8 * b2, 8)],
                        out_hbm.at[pl.ds(row0, 8)], wsem)
            drain_writes()
            drain_writes()
            return carry

        lax.fori_loop(0, NST, body, 0)

    return k2


def kernel(x, table):
    tT = table.T
    tail8 = table[:, 92:100].T
    xT = x.T.astype(jnp.int32)
    tpad = _build_k1()(tT, tail8)
    outp = _build_k2()(xT, tpad)
    out = outp[:, :DIM]
    return out.reshape(4096, 200, DIM)
